# baseline (speedup 1.0000x reference)
import jax
import jax.numpy as jnp
import numpy as np
from jax.experimental import pallas as pl
from jax.experimental.pallas import tpu as pltpu

_B, _T, _N = 128, 512, 384
_ROWS = _B * _T
_RB = 2048
_GRID = _ROWS // _RB
_P_MASK = np.float32(0.3)
_P_ZERO = np.float32(0.8)
_P_RAND = np.float32(0.1)

_ROT = ((13, 15, 26, 6), (17, 29, 16, 24))


def _threefry_bits(k0, k1, cnt):
    ks2 = k0 ^ k1 ^ np.uint32(0x1BD11BDA)
    x0 = jnp.zeros_like(cnt) + k0
    x1 = cnt + k1
    ks = (k0, k1, ks2)
    for i in range(5):
        for r in _ROT[i % 2]:
            x0 = x0 + x1
            x1 = (x1 << np.uint32(r)) | (x1 >> np.uint32(32 - r))
            x1 = x1 ^ x0
        x0 = x0 + ks[(i + 1) % 3]
        x1 = x1 + ks[(i + 2) % 3] + np.uint32(i + 1)
    return x0 ^ x1


def _bits_to_unif(bits):
    fb = (bits >> np.uint32(9)) | np.uint32(0x3F800000)
    return jax.lax.bitcast_convert_type(fb, jnp.float32) - np.float32(1.0)


def _row_counters(step):
    r = jax.lax.broadcasted_iota(jnp.uint32, (_RB, _N), 0)
    n = jax.lax.broadcasted_iota(jnp.uint32, (_RB, _N), 1)
    base = (step * np.uint32(_RB * _N))
    return base + r * np.uint32(_N) + n


def _mask_rows(step, k0, k1):
    r = jax.lax.broadcasted_iota(jnp.uint32, (_RB, 128), 0)
    cnt = step * np.uint32(_RB) + r
    return _bits_to_unif(_threefry_bits(k0, k1, cnt))


def _pass1_kernel(keys_ref, spikes_ref, code_ref, max_ref):
    step = pl.program_id(0).astype(jnp.uint32)
    k_mask0 = keys_ref[0, 0]
    k_mask1 = keys_ref[0, 1]
    k_zero0 = keys_ref[1, 0]
    k_zero1 = keys_ref[1, 1]

    u128 = _mask_rows(step, k_mask0, k_mask1)
    mask = jnp.concatenate([u128, u128, u128], axis=1) < _P_MASK

    u_zero = _bits_to_unif(_threefry_bits(k_zero0, k_zero1, _row_counters(step)))
    zero_idx = (u_zero < _P_ZERO) & mask

    x = spikes_ref[...]
    zeroed = jnp.where(zero_idx, np.float32(0.0), x)
    code = jnp.where(zero_idx, np.int32(1), jnp.where(mask, np.int32(2), np.int32(0)))
    code_ref[...] = code.astype(jnp.int8)

    bmax = jnp.max(zeroed)

    @pl.when(pl.program_id(0) == 0)
    def _init():
        max_ref[0, 0] = np.float32(-np.inf)

    max_ref[0, 0] = jnp.maximum(max_ref[0, 0], bmax)


def _pass2_kernel(keys_ref, max_ref, spikes_ref, code_ref, out_ref, tgt_ref):
    step = pl.program_id(0).astype(jnp.uint32)
    k_rand0 = keys_ref[2, 0]
    k_rand1 = keys_ref[2, 1]
    k_vals0 = keys_ref[3, 0]
    k_vals1 = keys_ref[3, 1]

    cnt = _row_counters(step)
    u_rand = _bits_to_unif(_threefry_bits(k_rand0, k_rand1, cnt))
    u_vals = _bits_to_unif(_threefry_bits(k_vals0, k_vals1, cnt))

    code = code_ref[...].astype(jnp.int32)
    x = spikes_ref[...]
    zeroed = jnp.where(code == 1, np.float32(0.0), x)
    rand_idx = (code == 2) & (u_rand < _P_RAND)
    repl = max_ref[0, 0] * u_vals
    out_ref[...] = jnp.where(rand_idx, repl, zeroed)
    tgt_ref[...] = (code != 0).astype(jnp.int32)


@jax.jit
def kernel(spikes):
    key = jax.random.key(42)
    subkeys = jax.random.key_data(jax.random.split(key, 4)).astype(jnp.uint32)

    flat = spikes.reshape(_ROWS, _N)

    code, maxval = pl.pallas_call(
        _pass1_kernel,
        grid=(_GRID,),
        in_specs=[
            pl.BlockSpec(memory_space=pltpu.SMEM),
            pl.BlockSpec((_RB, _N), lambda i: (i, 0)),
        ],
        out_specs=[
            pl.BlockSpec((_RB, _N), lambda i: (i, 0)),
            pl.BlockSpec(memory_space=pltpu.SMEM),
        ],
        out_shape=[
            jax.ShapeDtypeStruct((_ROWS, _N), jnp.int8),
            jax.ShapeDtypeStruct((1, 1), jnp.float32),
        ],
    )(subkeys, flat)

    out, tgt = pl.pallas_call(
        _pass2_kernel,
        grid=(_GRID,),
        in_specs=[
            pl.BlockSpec(memory_space=pltpu.SMEM),
            pl.BlockSpec(memory_space=pltpu.SMEM),
            pl.BlockSpec((_RB, _N), lambda i: (i, 0)),
            pl.BlockSpec((_RB, _N), lambda i: (i, 0)),
        ],
        out_specs=[
            pl.BlockSpec((_RB, _N), lambda i: (i, 0)),
            pl.BlockSpec((_RB, _N), lambda i: (i, 0)),
        ],
        out_shape=[
            jax.ShapeDtypeStruct((_ROWS, _N), jnp.float32),
            jax.ShapeDtypeStruct((_ROWS, _N), jnp.int32),
        ],
    )(subkeys, maxval, flat, code)

    return (out.reshape(_B, _T, _N),
            tgt.reshape(_B, _T, _N).astype(jnp.int64))
